# baseline (speedup 1.0000x reference)
import functools

import jax
import jax.numpy as jnp
import numpy as np
from jax import lax
from jax.experimental import pallas as pl
from jax.experimental.pallas import tpu as pltpu
from jax.experimental.pallas import tpu_sc as plsc

_B = 16
_N = 50000
_E = 1600000
_NQ = 8
_NV = 4
_NW = 32
_ESHARD = _E // _NQ
_CHUNK = 1600
_NBUF = 5
_NCHUNKS = _ESHARD // _CHUNK
_GROUPS = _CHUNK // 16
_HIMASK = np.int32(-65536)
_LOMASK = np.int32(0xFFFF)


def _sc_energy(xp, J, ei, ej):
    mesh = plsc.VectorSubcoreMesh(core_axis_name="c", subcore_axis_name="s")

    @functools.partial(
        pl.kernel,
        out_type=jax.ShapeDtypeStruct((_NW, 4, 16), jnp.float32),
        mesh=mesh,
        compiler_params=pltpu.CompilerParams(needs_layout_passes=False),
        scratch_types=[
            pltpu.VMEM((_N,), jnp.int32),
            pltpu.VMEM((_N,), jnp.int32),
        ]
        + [pltpu.VMEM((_CHUNK,), jnp.int32) for _ in range(_NBUF)]
        + [pltpu.VMEM((_CHUNK,), jnp.int32) for _ in range(_NBUF)]
        + [pltpu.VMEM((_CHUNK,), jnp.float32) for _ in range(_NBUF)]
        + [pltpu.VMEM((4, 16), jnp.float32)]
        + [pltpu.SemaphoreType.DMA for _ in range(_NBUF + 1)],
    )
    def body(xp_hbm, j_hbm, ei_hbm, ej_hbm, out_hbm, pk0, pk1, *rest):
        iis = rest[0:_NBUF]
        jjs = rest[_NBUF : 2 * _NBUF]
        jws = rest[2 * _NBUF : 3 * _NBUF]
        ov = rest[3 * _NBUF]
        sems = rest[3 * _NBUF + 1 : 4 * _NBUF + 1]
        rsem = rest[4 * _NBUF + 1]
        c = lax.axis_index("c")
        s = lax.axis_index("s")
        v = s % _NV
        q = (s // _NV) * 2 + c
        wid = v * _NQ + q
        ebase = q * _ESHARD
        bufs = tuple((iis[p], jjs[p], jws[p], sems[p]) for p in range(_NBUF))

        def start(buf, ci):
            bii, bjj, bjw, sem = buf
            off = ebase + ci * _CHUNK
            pltpu.async_copy(ei_hbm.at[pl.ds(off, _CHUNK)], bii, sem)
            pltpu.async_copy(ej_hbm.at[pl.ds(off, _CHUNK)], bjj, sem)
            pltpu.async_copy(j_hbm.at[pl.ds(off, _CHUNK)], bjw, sem)

        def wait(buf):
            bii, bjj, bjw, sem = buf
            pltpu.make_async_copy(ei_hbm.at[pl.ds(0, _CHUNK)], bii, sem).wait()
            pltpu.make_async_copy(ej_hbm.at[pl.ds(0, _CHUNK)], bjj, sem).wait()
            pltpu.make_async_copy(j_hbm.at[pl.ds(0, _CHUNK)], bjw, sem).wait()

        for p in range(_NBUF - 1):
            start(bufs[p], p)
        pltpu.async_copy(xp_hbm.at[2 * v], pk0, rsem)
        pltpu.async_copy(xp_hbm.at[2 * v + 1], pk1, rsem)
        pltpu.make_async_copy(xp_hbm.at[2 * v], pk0, rsem).wait()
        pltpu.make_async_copy(xp_hbm.at[2 * v + 1], pk1, rsem).wait()

        def outer(ci, acc):
            accs = acc
            for p in range(_NBUF):
                buf = bufs[p]
                cur = ci + p
                wait(buf)

                @pl.when(cur + _NBUF - 1 < _NCHUNKS)
                def _():
                    start(bufs[(p + _NBUF - 1) % _NBUF], cur + _NBUF - 1)

                bii, bjj, bjw, _sem = buf

                def grp(k, a):
                    a0, a1, a2, a3 = a
                    base = k * 16
                    iv = bii[pl.ds(base, 16)]
                    jv = bjj[pl.ds(base, 16)]
                    w = bjw[pl.ds(base, 16)]
                    gi0 = plsc.load_gather(pk0, [iv])
                    gj0 = plsc.load_gather(pk0, [jv])
                    gi1 = plsc.load_gather(pk1, [iv])
                    gj1 = plsc.load_gather(pk1, [jv])

                    def hi(g):
                        return plsc.bitcast(lax.bitwise_and(g, _HIMASK), jnp.float32)

                    def lo(g):
                        return plsc.bitcast(lax.shift_left(g, 16), jnp.float32)

                    a0 = a0 + hi(gi0) * hi(gj0) * w
                    a1 = a1 + lo(gi0) * lo(gj0) * w
                    a2 = a2 + hi(gi1) * hi(gj1) * w
                    a3 = a3 + lo(gi1) * lo(gj1) * w
                    return (a0, a1, a2, a3)

                accs = pl.loop(0, _GROUPS, init_carry=accs, unroll=10)(grp)
            return accs

        z = jnp.zeros((16,), jnp.float32)
        a0, a1, a2, a3 = pl.loop(0, _NCHUNKS, step=_NBUF, init_carry=(z, z, z, z))(outer)
        ov[0] = a0
        ov[1] = a1
        ov[2] = a2
        ov[3] = a3
        pltpu.sync_copy(ov, out_hbm.at[wid])

    return body(xp, J, ei, ej)


def _tc_prep(x, h):
    def pack_body(x_ref, xp_ref):
        u = lax.bitcast_convert_type(x_ref[...], jnp.uint32)

        def rn(v):
            return (v + jnp.uint32(0x7FFF) + ((v >> 16) & jnp.uint32(1))) & jnp.uint32(
                0xFFFF0000
            )

        xp_ref[...] = lax.bitcast_convert_type(
            rn(u[0:8]) | (rn(u[8:16]) >> 16), jnp.int32
        )

    def hx_body(x_ref, h_ref, hx_ref):
        hx_ref[...] = jnp.sum(x_ref[...] * h_ref[...], axis=1, keepdims=True)

    xp = pl.pallas_call(
        pack_body, out_shape=jax.ShapeDtypeStruct((8, _N), jnp.int32)
    )(x)
    hx = pl.pallas_call(
        hx_body, out_shape=jax.ShapeDtypeStruct((_B, 1), jnp.float32)
    )(x, h.reshape(1, _N))
    return xp, hx


_ROW_ORDER = np.argsort(
    np.array([[2 * v, 2 * v + 8, 2 * v + 1, 2 * v + 9] for v in range(_NV)]).reshape(-1)
)


def kernel(x, h, J, edge_idx_i, edge_idx_j):
    xp, hx = _tc_prep(x, h)
    ei = edge_idx_i.astype(jnp.int32)
    ej = edge_idx_j.astype(jnp.int32)
    parts = _sc_energy(xp, J, ei, ej)
    r = parts.reshape(_NV, _NQ, 4, 16).sum(axis=(1, 3)).reshape(_B)
    return r[_ROW_ORDER] + hx[:, 0]


# speedup vs baseline: 1.0401x; 1.0401x over previous
import functools

import jax
import jax.numpy as jnp
import numpy as np
from jax import lax
from jax.experimental import pallas as pl
from jax.experimental.pallas import tpu as pltpu
from jax.experimental.pallas import tpu_sc as plsc

_B = 16
_N = 50000
_E = 1600000
_NQ = 8
_NV = 4
_NW = 32
_ESHARD = _E // _NQ
_CHUNK = 2000
_NBUF = 4
_NCHUNKS = _ESHARD // _CHUNK
_GROUPS = _CHUNK // 16
_HIMASK = np.int32(-65536)
_LOMASK = np.int32(0xFFFF)


def _sc_energy(xp, J, ei, ej):
    mesh = plsc.VectorSubcoreMesh(core_axis_name="c", subcore_axis_name="s")

    @functools.partial(
        pl.kernel,
        out_type=jax.ShapeDtypeStruct((_NW, 4, 16), jnp.float32),
        mesh=mesh,
        compiler_params=pltpu.CompilerParams(needs_layout_passes=False),
        scratch_types=[
            pltpu.VMEM((_N,), jnp.int32),
            pltpu.VMEM((_N,), jnp.int32),
        ]
        + [pltpu.VMEM((_CHUNK,), jnp.int32) for _ in range(_NBUF)]
        + [pltpu.VMEM((_CHUNK,), jnp.int32) for _ in range(_NBUF)]
        + [pltpu.VMEM((_CHUNK,), jnp.float32) for _ in range(_NBUF)]
        + [pltpu.VMEM((4, 16), jnp.float32)]
        + [pltpu.SemaphoreType.DMA for _ in range(_NBUF + 1)],
    )
    def body(xp_hbm, j_hbm, ei_hbm, ej_hbm, out_hbm, pk0, pk1, *rest):
        iis = rest[0:_NBUF]
        jjs = rest[_NBUF : 2 * _NBUF]
        jws = rest[2 * _NBUF : 3 * _NBUF]
        ov = rest[3 * _NBUF]
        sems = rest[3 * _NBUF + 1 : 4 * _NBUF + 1]
        rsem = rest[4 * _NBUF + 1]
        c = lax.axis_index("c")
        s = lax.axis_index("s")
        v = s % _NV
        q = (s // _NV) * 2 + c
        wid = v * _NQ + q
        ebase = q * _ESHARD
        bufs = tuple((iis[p], jjs[p], jws[p], sems[p]) for p in range(_NBUF))

        def start(buf, ci):
            bii, bjj, bjw, sem = buf
            off = ebase + ci * _CHUNK
            pltpu.async_copy(ei_hbm.at[pl.ds(off, _CHUNK)], bii, sem)
            pltpu.async_copy(ej_hbm.at[pl.ds(off, _CHUNK)], bjj, sem)
            pltpu.async_copy(j_hbm.at[pl.ds(off, _CHUNK)], bjw, sem)

        def wait(buf):
            bii, bjj, bjw, sem = buf
            pltpu.make_async_copy(ei_hbm.at[pl.ds(0, _CHUNK)], bii, sem).wait()
            pltpu.make_async_copy(ej_hbm.at[pl.ds(0, _CHUNK)], bjj, sem).wait()
            pltpu.make_async_copy(j_hbm.at[pl.ds(0, _CHUNK)], bjw, sem).wait()

        for p in range(_NBUF - 1):
            start(bufs[p], p)
        pltpu.async_copy(xp_hbm.at[2 * v], pk0, rsem)
        pltpu.async_copy(xp_hbm.at[2 * v + 1], pk1, rsem)
        pltpu.make_async_copy(xp_hbm.at[2 * v], pk0, rsem).wait()
        pltpu.make_async_copy(xp_hbm.at[2 * v + 1], pk1, rsem).wait()

        def outer(ci, acc):
            accs = acc
            for p in range(_NBUF):
                buf = bufs[p]
                cur = ci + p
                wait(buf)

                @pl.when(cur + _NBUF - 1 < _NCHUNKS)
                def _():
                    start(bufs[(p + _NBUF - 1) % _NBUF], cur + _NBUF - 1)

                bii, bjj, bjw, _sem = buf

                def grp(k, a):
                    a0, a1, a2, a3 = a
                    base = k * 16
                    iv = bii[pl.ds(base, 16)]
                    jv = bjj[pl.ds(base, 16)]
                    w = bjw[pl.ds(base, 16)]
                    gi0 = plsc.load_gather(pk0, [iv])
                    gj0 = plsc.load_gather(pk0, [jv])
                    gi1 = plsc.load_gather(pk1, [iv])
                    gj1 = plsc.load_gather(pk1, [jv])

                    def hi(g):
                        return plsc.bitcast(lax.bitwise_and(g, _HIMASK), jnp.float32)

                    def lo(g):
                        return plsc.bitcast(lax.shift_left(g, 16), jnp.float32)

                    a0 = a0 + hi(gi0) * hi(gj0) * w
                    a1 = a1 + lo(gi0) * lo(gj0) * w
                    a2 = a2 + hi(gi1) * hi(gj1) * w
                    a3 = a3 + lo(gi1) * lo(gj1) * w
                    return (a0, a1, a2, a3)

                accs = pl.loop(0, _GROUPS, init_carry=accs, unroll=5)(grp)
            return accs

        z = jnp.zeros((16,), jnp.float32)
        a0, a1, a2, a3 = pl.loop(0, _NCHUNKS, step=_NBUF, init_carry=(z, z, z, z))(outer)
        ov[0] = a0
        ov[1] = a1
        ov[2] = a2
        ov[3] = a3
        pltpu.sync_copy(ov, out_hbm.at[wid])

    return body(xp, J, ei, ej)


def _tc_prep(x, h):
    def pack_body(x_ref, xp_ref):
        u = lax.bitcast_convert_type(x_ref[...], jnp.uint32)

        def rn(v):
            return (v + jnp.uint32(0x7FFF) + ((v >> 16) & jnp.uint32(1))) & jnp.uint32(
                0xFFFF0000
            )

        xp_ref[...] = lax.bitcast_convert_type(
            rn(u[0:8]) | (rn(u[8:16]) >> 16), jnp.int32
        )

    def hx_body(x_ref, h_ref, hx_ref):
        hx_ref[...] = jnp.sum(x_ref[...] * h_ref[...], axis=1, keepdims=True)

    xp = pl.pallas_call(
        pack_body, out_shape=jax.ShapeDtypeStruct((8, _N), jnp.int32)
    )(x)
    hx = pl.pallas_call(
        hx_body, out_shape=jax.ShapeDtypeStruct((_B, 1), jnp.float32)
    )(x, h.reshape(1, _N))
    return xp, hx


_ROW_ORDER = np.argsort(
    np.array([[2 * v, 2 * v + 8, 2 * v + 1, 2 * v + 9] for v in range(_NV)]).reshape(-1)
)


def kernel(x, h, J, edge_idx_i, edge_idx_j):
    xp, hx = _tc_prep(x, h)
    ei = edge_idx_i.astype(jnp.int32)
    ej = edge_idx_j.astype(jnp.int32)
    parts = _sc_energy(xp, J, ei, ej)
    r = parts.reshape(_NV, _NQ, 4, 16).sum(axis=(1, 3)).reshape(_B)
    return r[_ROW_ORDER] + hx[:, 0]
